# baseline (speedup 1.0000x reference)
import jax
import jax.numpy as jnp
import numpy as np
from jax.experimental import pallas as pl
from jax.experimental.pallas import tpu as pltpu

_F32 = jnp.float32
_BF16 = jnp.bfloat16


def _lenet_body(x_ref, w1_ref, w2_ref, wf_ref, bf_ref, o_ref):
    tb = x_ref.shape[0]

    xv = x_ref[...].astype(_BF16)
    xp1 = jnp.concatenate(
        [jnp.ones((tb, 32, 1), _BF16), jnp.zeros((tb, 32, 1), _BF16),
         jnp.pad(xv, ((0, 0), (2, 2), (0, 0))),
         jnp.zeros((tb, 32, 2), _BF16)], axis=-1)
    lhs1 = jnp.concatenate([xp1[:, d:d + 28, :] for d in range(5)], axis=-1)
    y1 = jnp.dot(lhs1.reshape(tb * 28, 160), w1_ref[...],
                 preferred_element_type=_F32)
    r1 = jnp.maximum(y1.reshape(tb, 28, 512), 0.0)

    mx = jnp.maximum(r1[:, :, 0:224], r1[:, :, 256:480])
    m4 = mx.reshape(tb, 14, 2, 224)
    p1 = jnp.maximum(m4[:, :, 0, :], m4[:, :, 1, :])

    xp2 = jnp.concatenate(
        [jnp.ones((tb, 18, 1), _BF16), jnp.zeros((tb, 18, 31), _BF16),
         jnp.pad(p1, ((0, 0), (2, 2), (0, 0))).astype(_BF16),
         jnp.zeros((tb, 18, 32), _BF16)], axis=-1)
    lhs2 = jnp.concatenate([xp2[:, d:d + 14, :] for d in range(5)], axis=-1)
    y2 = jnp.dot(lhs2.reshape(tb * 14, 1440), w2_ref[...],
                 preferred_element_type=_F32)
    r2 = jnp.maximum(y2.reshape(tb, 14, 512), 0.0)

    mx2 = jnp.maximum(r2[:, :, 0:224], r2[:, :, 256:480])
    m24 = mx2.reshape(tb, 7, 2, 224)
    p2 = jnp.maximum(m24[:, :, 0, :], m24[:, :, 1, :]).astype(_BF16)

    lhsf = jnp.concatenate([p2[:, hh, :] for hh in range(7)], axis=-1)
    logits = jnp.dot(lhsf, wf_ref[...], preferred_element_type=_F32)
    o_ref[...] = (logits + bf_ref[...])[:, :10]


def kernel(w1, b1, w2, b2, wf, bf, x):
    B = x.shape[0]
    TB = 128 if B % 128 == 0 else (8 if B % 8 == 0 else B)

    w1r = w1.reshape(5, 5, 16)
    dws = np.arange(5)[:, None]
    xs = np.arange(28)[None, :]
    W1 = jnp.zeros((5, 32, 2, 16, 16), _F32)
    W1 = W1.at[:, dws + xs, xs % 2, xs // 2, :].set(w1r[:, :, None, :])
    W1 = W1.at[:, 0].set(0.0)
    W1 = W1.at[2, 0].set(b1.reshape(1, 1, 16))
    W1 = W1.reshape(160, 512).astype(_BF16)

    w2r = w2.reshape(5, 5, 16, 32)
    xs2 = np.arange(14)[None, :]
    W2 = jnp.zeros((5, 18, 16, 2, 8, 32), _F32)
    W2 = W2.at[:, dws + xs2, :, xs2 % 2, xs2 // 2, :].set(
        jnp.transpose(w2r, (1, 0, 2, 3))[:, None])
    W2 = W2.at[:, 0, 0].set(0.0)
    W2 = W2.at[2, 0, 0].set(b2.reshape(1, 1, 32))
    W2 = W2.reshape(1440, 512).astype(_BF16)

    Wf = wf.reshape(1568, 128).astype(_BF16)

    x3 = x.reshape(B, 28, 28)

    return pl.pallas_call(
        _lenet_body,
        out_shape=jax.ShapeDtypeStruct((B, 10), _F32),
        grid=(B // TB,),
        in_specs=[
            pl.BlockSpec((TB, 28, 28), lambda i: (i, 0, 0)),
            pl.BlockSpec((160, 512), lambda i: (0, 0)),
            pl.BlockSpec((1440, 512), lambda i: (0, 0)),
            pl.BlockSpec((1568, 128), lambda i: (0, 0)),
            pl.BlockSpec((1, 128), lambda i: (0, 0)),
        ],
        out_specs=pl.BlockSpec((TB, 10), lambda i: (i, 0)),
        compiler_params=pltpu.CompilerParams(
            dimension_semantics=("parallel",),
            vmem_limit_bytes=64 * 1024 * 1024,
        ),
    )(x3, W1, W2, Wf, bf)


# speedup vs baseline: 1.0620x; 1.0620x over previous
import jax
import jax.numpy as jnp
import numpy as np
from jax.experimental import pallas as pl
from jax.experimental.pallas import tpu as pltpu

_F32 = jnp.float32
_BF16 = jnp.bfloat16


def _lenet_body(x_ref, w1_ref, w2_ref, wf_ref, bf_ref, o_ref):
    tb = x_ref.shape[0]

    xv = x_ref[...].astype(_BF16)
    xp1 = jnp.concatenate(
        [jnp.pad(xv, ((0, 0), (2, 2), (0, 0))),
         jnp.zeros((tb, 32, 2), _BF16),
         jnp.ones((tb, 32, 1), _BF16),
         jnp.zeros((tb, 32, 1), _BF16)], axis=-1)
    lhs1 = jnp.concatenate([xp1[:, d:d + 28, :] for d in range(5)], axis=-1)
    y1 = jnp.dot(lhs1.reshape(tb * 28, 160), w1_ref[...],
                 preferred_element_type=_F32)
    r1 = jnp.maximum(y1.reshape(tb, 28, 512), 0.0)

    mx = jnp.maximum(r1[:, :, 0:256], r1[:, :, 256:512])
    m4 = mx.reshape(tb, 14, 2, 256)
    p1 = jnp.maximum(m4[:, :, 0, :], m4[:, :, 1, :])

    xp2 = jnp.concatenate(
        [jnp.pad(p1, ((0, 0), (2, 2), (0, 0))).astype(_BF16),
         jnp.ones((tb, 18, 1), _BF16),
         jnp.zeros((tb, 18, 31), _BF16)], axis=-1)
    lhs2 = jnp.concatenate([xp2[:, d:d + 14, :] for d in range(5)], axis=-1)
    y2 = jnp.dot(lhs2.reshape(tb * 14, 1440), w2_ref[...],
                 preferred_element_type=_F32)
    r2 = jnp.maximum(y2.reshape(tb, 14, 512), 0.0)

    mx2 = jnp.maximum(r2[:, :, 0:256], r2[:, :, 256:512])
    m24 = mx2.reshape(tb, 7, 2, 256)
    p2 = jnp.maximum(m24[:, :, 0, :], m24[:, :, 1, :]).astype(_BF16)

    lhsf = p2.reshape(tb, 1792)
    logits = jnp.dot(lhsf, wf_ref[...], preferred_element_type=_F32)
    o_ref[...] = (logits + bf_ref[...])[:, :10]


def kernel(w1, b1, w2, b2, wf, bf, x):
    B = x.shape[0]
    TB = 64 if B % 64 == 0 else (8 if B % 8 == 0 else B)

    w1r = w1.reshape(5, 5, 16)
    dws = np.arange(5)[:, None]
    xs = np.arange(28)[None, :]
    W1c = jnp.zeros((5, 32, 2, 16, 16), _F32)
    W1c = W1c.at[:, dws + xs, xs % 2, xs // 2, :].set(w1r[:, :, None, :])
    W1b = jnp.zeros((5, 1, 2, 16, 16), _F32).at[2, 0].set(b1.reshape(1, 1, 16))
    W1 = jnp.concatenate(
        [W1c[:, 2:30], jnp.zeros((5, 2, 2, 16, 16), _F32), W1b,
         jnp.zeros((5, 1, 2, 16, 16), _F32)], axis=1)
    W1 = W1.reshape(160, 512).astype(_BF16)

    w2r = w2.reshape(5, 5, 16, 32)
    xs2 = np.arange(14)[None, :]
    W2c = jnp.zeros((5, 18, 16, 2, 8, 32), _F32)
    W2c = W2c.at[:, dws + xs2, :, xs2 % 2, xs2 // 2, :].set(
        jnp.transpose(w2r, (1, 0, 2, 3))[:, None])
    W2b = jnp.zeros((5, 1, 16, 2, 8, 32), _F32).at[2, 0, 0].set(
        b2.reshape(1, 1, 32))
    W2 = jnp.concatenate(
        [W2c[:, 2:16], jnp.zeros((5, 2, 16, 2, 8, 32), _F32), W2b,
         jnp.zeros((5, 1, 16, 2, 8, 32), _F32)], axis=1)
    W2 = W2.reshape(1440, 512).astype(_BF16)

    Wf = jnp.pad(wf.reshape(7, 224, 128), ((0, 0), (0, 32), (0, 0)))
    Wf = Wf.reshape(1792, 128).astype(_BF16)

    x3 = x.reshape(B, 28, 28)

    return pl.pallas_call(
        _lenet_body,
        out_shape=jax.ShapeDtypeStruct((B, 10), _F32),
        grid=(B // TB,),
        in_specs=[
            pl.BlockSpec((TB, 28, 28), lambda i: (i, 0, 0)),
            pl.BlockSpec((160, 512), lambda i: (0, 0)),
            pl.BlockSpec((1440, 512), lambda i: (0, 0)),
            pl.BlockSpec((1792, 128), lambda i: (0, 0)),
            pl.BlockSpec((1, 128), lambda i: (0, 0)),
        ],
        out_specs=pl.BlockSpec((TB, 10), lambda i: (i, 0)),
        compiler_params=pltpu.CompilerParams(
            dimension_semantics=("parallel",),
            vmem_limit_bytes=64 * 1024 * 1024,
        ),
    )(x3, W1, W2, Wf, bf)


# speedup vs baseline: 1.1860x; 1.1167x over previous
import jax
import jax.numpy as jnp
import numpy as np
from jax.experimental import pallas as pl
from jax.experimental.pallas import tpu as pltpu

_F32 = jnp.float32
_BF16 = jnp.bfloat16


def _lenet_body(x_ref, w1_ref, w2_ref, wf_ref, bf_ref, o_ref):
    tb = x_ref.shape[0]

    xv = x_ref[...].astype(_BF16)
    xp1 = jnp.concatenate(
        [jnp.pad(xv, ((0, 0), (2, 2), (0, 0))),
         jnp.zeros((tb, 32, 2), _BF16),
         jnp.ones((tb, 32, 1), _BF16),
         jnp.zeros((tb, 32, 1), _BF16)], axis=-1)
    lhs1 = jnp.concatenate([xp1[:, d:d + 28, :] for d in range(5)], axis=-1)
    y1 = jnp.dot(lhs1.reshape(tb * 28, 160), w1_ref[...],
                 preferred_element_type=_F32)
    r1 = jnp.maximum(y1.reshape(tb, 28, 512), 0.0)

    mx = jnp.maximum(r1[:, :, 0:256], r1[:, :, 256:512])
    m4 = mx.reshape(tb, 14, 2, 256)
    p1 = jnp.maximum(m4[:, :, 0, :], m4[:, :, 1, :])

    xp2 = jnp.concatenate(
        [jnp.pad(p1, ((0, 0), (2, 2), (0, 0))),
         jnp.ones((tb, 18, 1), _F32),
         jnp.zeros((tb, 18, 31), _F32)], axis=-1)
    lhs2 = jnp.concatenate([xp2[:, d:d + 14, :] for d in range(5)], axis=-1)
    y2 = jnp.dot(lhs2.reshape(tb * 14, 1440), w2_ref[...],
                 preferred_element_type=_F32)
    r2 = jnp.maximum(y2.reshape(tb, 14, 512), 0.0)

    mx2 = jnp.maximum(r2[:, :, 0:256], r2[:, :, 256:512])
    m24 = mx2.reshape(tb, 7, 2, 256)
    p2 = jnp.maximum(m24[:, :, 0, :], m24[:, :, 1, :])

    lhsf = p2.reshape(tb, 1792)
    logits = jnp.dot(lhsf, wf_ref[...], preferred_element_type=_F32)
    o_ref[...] = (logits + bf_ref[...])[:, :10]


def kernel(w1, b1, w2, b2, wf, bf, x):
    B = x.shape[0]
    TB = 64 if B % 64 == 0 else (8 if B % 8 == 0 else B)

    w1r = w1.reshape(5, 5, 16)
    dws = np.arange(5)[:, None]
    xs = np.arange(28)[None, :]
    W1c = jnp.zeros((5, 32, 2, 16, 16), _F32)
    W1c = W1c.at[:, dws + xs, xs % 2, xs // 2, :].set(w1r[:, :, None, :])
    W1b = jnp.zeros((5, 1, 2, 16, 16), _F32).at[2, 0].set(b1.reshape(1, 1, 16))
    W1 = jnp.concatenate(
        [W1c[:, 2:30], jnp.zeros((5, 2, 2, 16, 16), _F32), W1b,
         jnp.zeros((5, 1, 2, 16, 16), _F32)], axis=1)
    W1 = W1.reshape(160, 512).astype(_BF16)

    w2r = w2.reshape(5, 5, 16, 32)
    xs2 = np.arange(14)[None, :]
    W2c = jnp.zeros((5, 18, 16, 2, 8, 32), _F32)
    W2c = W2c.at[:, dws + xs2, :, xs2 % 2, xs2 // 2, :].set(
        jnp.transpose(w2r, (1, 0, 2, 3))[:, None])
    W2b = jnp.zeros((5, 1, 16, 2, 8, 32), _F32).at[2, 0, 0].set(
        b2.reshape(1, 1, 32))
    W2 = jnp.concatenate(
        [W2c[:, 2:16], jnp.zeros((5, 2, 16, 2, 8, 32), _F32), W2b,
         jnp.zeros((5, 1, 16, 2, 8, 32), _F32)], axis=1)
    W2 = W2.reshape(1440, 512)

    Wf = jnp.pad(wf.reshape(7, 224, 128), ((0, 0), (0, 32), (0, 0)))
    Wf = Wf.reshape(1792, 128)

    x3 = x.reshape(B, 28, 28)

    return pl.pallas_call(
        _lenet_body,
        out_shape=jax.ShapeDtypeStruct((B, 10), _F32),
        grid=(B // TB,),
        in_specs=[
            pl.BlockSpec((TB, 28, 28), lambda i: (i, 0, 0)),
            pl.BlockSpec((160, 512), lambda i: (0, 0)),
            pl.BlockSpec((1440, 512), lambda i: (0, 0)),
            pl.BlockSpec((1792, 128), lambda i: (0, 0)),
            pl.BlockSpec((1, 128), lambda i: (0, 0)),
        ],
        out_specs=pl.BlockSpec((TB, 10), lambda i: (i, 0)),
        compiler_params=pltpu.CompilerParams(
            dimension_semantics=("parallel",),
            vmem_limit_bytes=64 * 1024 * 1024,
        ),
    )(x3, W1, W2, Wf, bf)
